# baseline (speedup 1.0000x reference)
import functools

import jax
import jax.numpy as jnp
from jax import lax
from jax.experimental import pallas as pl
from jax.experimental.pallas import tpu as pltpu
from jax.experimental.pallas import tpu_sc as plsc

N_POINTS = 262144
N_LEVELS = 16
N_FEATS = 2
TS = 2 ** 19
MASK = TS - 1
PR1 = -1640531535
PR2 = 805459861
NC, NS = 2, 16
NW = NC * NS
PPW = N_POINTS // NW
CHUNK = 512
NCHUNK = PPW // CHUNK
G = CHUNK // 16
OUT_D = N_LEVELS * N_FEATS


def _sc_embed(xt, ftab, gs):
    mesh = plsc.VectorSubcoreMesh(
        core_axis_name="c", subcore_axis_name="s",
        num_cores=NC, num_subcores=NS)

    @functools.partial(
        pl.kernel,
        out_type=jax.ShapeDtypeStruct((N_POINTS, OUT_D), jnp.float32),
        mesh=mesh,
        compiler_params=pltpu.CompilerParams(
            needs_layout_passes=False, use_tc_tiling_on_sc=False),
        scratch_types=[
            pltpu.VMEM((CHUNK * 3,), jnp.float32),
            pltpu.VMEM((CHUNK * 8,), jnp.int32),
            pltpu.VMEM((CHUNK * 8,), jnp.int32),
            pltpu.VMEM((CHUNK * 8, 8), jnp.float32),
            pltpu.VMEM((CHUNK * 8, 8), jnp.float32),
            pltpu.VMEM((CHUNK, OUT_D), jnp.float32),
            pltpu.VMEM((16, 16), jnp.float32),
            pltpu.SemaphoreType.DMA,
            pltpu.SemaphoreType.DMA,
        ],
    )
    def body(xt_hbm, ftab_hbm, gs_hbm, out_hbm,
             xyz, ia, ib, ra, rb, ob, gsv, sa, sb):
        wid = lax.axis_index("s") * NC + lax.axis_index("c")
        base_w = wid * PPW
        pltpu.sync_copy(gs_hbm, gsv)
        iota = lax.iota(jnp.int32, 16)
        lane8 = iota * 8
        lane8c = [lane8 + c for c in range(8)]
        iota3 = iota * 3

        def gs_bcast(l):
            return gsv[l]
        idxbufs = [ia, ib]
        rowbufs = [ra, rb]
        sems = [sa, sb]
        zero_f = jnp.float32(0.0)
        one_f = jnp.float32(1.0)

        def load_xyz(g):
            p3 = g * 48 + iota3
            x = plsc.load_gather(xyz, [p3])
            y = plsc.load_gather(xyz, [p3 + 1])
            z = plsc.load_gather(xyz, [p3 + 2])
            return x, y, z

        def corner_hashes(x, y, z, gsl):
            xi, yi, zi = corner_idx(x, y, z, gsl)
            hx = (xi, xi + 1)
            y0 = yi * jnp.int32(PR1)
            hy = (y0, y0 + jnp.int32(PR1))
            z0 = zi * jnp.int32(PR2)
            hz = (z0, z0 + jnp.int32(PR2))
            hs = []
            for c in range(8):
                i, j, k = c >> 2, (c >> 1) & 1, c & 1
                hs.append((hx[i] ^ hy[j] ^ hz[k]) & MASK)
            return xi, yi, zi, hs

        def corner_idx(x, y, z, gsl):
            xc = jnp.minimum(jnp.maximum(x, zero_f), one_f)
            yc = jnp.minimum(jnp.maximum(y, zero_f), one_f)
            zc = jnp.minimum(jnp.maximum(z, zero_f), one_f)
            xi = (xc / gsl).astype(jnp.int32)
            yi = (yc / gsl).astype(jnp.int32)
            zi = (zc / gsl).astype(jnp.int32)
            return xi, yi, zi

        def phase_a(l, g):
            x, y, z = load_xyz(g)
            _, _, _, hs = corner_hashes(x, y, z, gs_bcast(l))
            pos0 = g * 128
            loff = l * TS
            dst = idxbufs[l % 2]
            for c in range(8):
                blk = lax.shift_right_logical(hs[c] + loff, 2)
                plsc.store_scatter(dst, [pos0 + lane8c[c]], blk)

        def phase_c(l, g):
            x, y, z = load_xyz(g)
            gsl = gs_bcast(l)
            xi, yi, zi, hs = corner_hashes(x, y, z, gsl)
            wx = (x - xi.astype(jnp.float32) * gsl) / gsl
            wy = (y - yi.astype(jnp.float32) * gsl) / gsl
            wz = (z - zi.astype(jnp.float32) * gsl) / gsl
            pos0 = g * 128
            rows = rowbufs[l % 2]
            prow = g * 16 + iota
            offs = [(hs[c] & 3) * 2 for c in range(8)]
            for f in range(N_FEATS):
                e = [plsc.load_gather(rows, [pos0 + lane8c[c], offs[c] + f])
                     for c in range(8)]
                c00 = e[0] * (one_f - wx) + e[4] * wx
                c01 = e[1] * (one_f - wx) + e[5] * wx
                c10 = e[2] * (one_f - wx) + e[6] * wx
                c11 = e[3] * (one_f - wx) + e[7] * wx
                c0 = c00 * (one_f - wy) + c10 * wy
                c1 = c01 * (one_f - wy) + c11 * wy
                r = c0 * (one_f - wz) + c1 * wz
                plsc.store_scatter(ob, [prow, jnp.full((16,), 2 * l + f, jnp.int32)], r)

        def chunk_body(ci, carry):
            cb = base_w + ci * CHUNK
            pltpu.sync_copy(xt_hbm.at[pl.ds(cb * 3, CHUNK * 3)], xyz)
            lax.fori_loop(0, G, lambda g, _: phase_a(0, g), None)
            cps = [None] * N_LEVELS
            cps[0] = pltpu.async_copy(ftab_hbm.at[idxbufs[0]], rowbufs[0], sems[0])
            for l in range(N_LEVELS):
                if l + 1 < N_LEVELS:
                    lax.fori_loop(0, G, lambda g, _, l=l: phase_a(l + 1, g), None)
                    cps[l + 1] = pltpu.async_copy(
                        ftab_hbm.at[idxbufs[(l + 1) % 2]],
                        rowbufs[(l + 1) % 2], sems[(l + 1) % 2])
                cps[l].wait()
                lax.fori_loop(0, G, lambda g, _, l=l: phase_c(l, g), None)
            pltpu.sync_copy(ob, out_hbm.at[pl.ds(cb, CHUNK)])
            return carry

        lax.fori_loop(0, NCHUNK, chunk_body, None)

    return body(xt, ftab, gs)


def kernel(x, tables):
    b = jnp.exp((jnp.log(jnp.float32(512.0)) - jnp.log(jnp.float32(16.0)))
                / (N_LEVELS - 1))
    res = jnp.stack([jnp.floor(jnp.float32(16.0) * b ** i)
                     for i in range(N_LEVELS)])
    gs = (jnp.float32(1.0) - jnp.float32(0.0)) / res
    gs_rep = jnp.tile(gs[:, None], (1, 16))
    scale = jnp.float32(1.0) + x[0, 0] * jnp.float32(0.0)
    xt = x.reshape(3 * N_POINTS) * scale
    ftab = tables.reshape(N_LEVELS * TS * N_FEATS // 8, 8) * scale
    out = _sc_embed(xt, ftab, gs_rep)
    keep_mask = x == jnp.maximum(jnp.minimum(x, 1.0), 0.0)
    keep_all = jnp.sum(keep_mask, axis=-1) == keep_mask.shape[-1]
    return out, keep_all


# speedup vs baseline: 7.3271x; 7.3271x over previous
import functools

import jax
import jax.numpy as jnp
from jax import lax
from jax.experimental import pallas as pl
from jax.experimental.pallas import tpu as pltpu
from jax.experimental.pallas import tpu_sc as plsc

N_POINTS = 262144
N_LEVELS = 16
N_FEATS = 2
TS = 2 ** 19
MASK = TS - 1
PR1 = -1640531535
PR2 = 805459861
NC, NS = 2, 16
NW = NC * NS
PPW = N_POINTS // NW
CHUNK = 512
NCHUNK = PPW // CHUNK
G = CHUNK // 16
OUT_D = N_LEVELS * N_FEATS


TWORDS = N_LEVELS * TS * N_FEATS
CONV_CHUNK = 16384
CONV_SLAB = TWORDS // NW
CONV_ITERS = CONV_SLAB // CONV_CHUNK


def _sc_relayout(tview):
    mesh = plsc.VectorSubcoreMesh(
        core_axis_name="c", subcore_axis_name="s",
        num_cores=NC, num_subcores=NS)

    @functools.partial(
        pl.kernel,
        out_type=jax.ShapeDtypeStruct((TWORDS,), jnp.float32),
        mesh=mesh,
        compiler_params=pltpu.CompilerParams(
            needs_layout_passes=False, use_tc_tiling_on_sc=False),
        scratch_types=[
            pltpu.VMEM((CONV_CHUNK,), jnp.float32),
            pltpu.VMEM((CONV_CHUNK,), jnp.float32),
        ],
    )
    def conv(in_hbm, out_hbm, vin, vout):
        wid = lax.axis_index("s") * NC + lax.axis_index("c")
        base = wid * CONV_SLAB
        iota = lax.iota(jnp.int32, 16)
        iota2 = iota * 2

        def blk(b, carry):
            b256 = b * 256
            for j in range(8):
                f0 = vin[pl.ds(b256 + j * 16, 16)]
                f1 = vin[pl.ds(b256 + 128 + j * 16, 16)]
                dst = b256 + j * 32 + iota2
                plsc.store_scatter(vout, [dst], f0)
                plsc.store_scatter(vout, [dst + 1], f1)
            return carry

        def it(ci, carry):
            off = base + ci * CONV_CHUNK
            pltpu.sync_copy(in_hbm.at[pl.ds(off, CONV_CHUNK)], vin)
            lax.fori_loop(0, CONV_CHUNK // 256, blk, None)
            pltpu.sync_copy(vout, out_hbm.at[pl.ds(off, CONV_CHUNK)])
            return carry

        lax.fori_loop(0, CONV_ITERS, it, None)

    return conv(tview)


def _sc_embed(xt, ftab, gs):
    mesh = plsc.VectorSubcoreMesh(
        core_axis_name="c", subcore_axis_name="s",
        num_cores=NC, num_subcores=NS)

    @functools.partial(
        pl.kernel,
        out_type=jax.ShapeDtypeStruct((N_POINTS, OUT_D), jnp.float32),
        mesh=mesh,
        compiler_params=pltpu.CompilerParams(
            needs_layout_passes=False, use_tc_tiling_on_sc=False),
        scratch_types=[
            pltpu.VMEM((CHUNK * 3,), jnp.float32),
            pltpu.VMEM((CHUNK * 8,), jnp.int32),
            pltpu.VMEM((CHUNK * 8,), jnp.int32),
            pltpu.VMEM((CHUNK * 8, 8), jnp.float32),
            pltpu.VMEM((CHUNK * 8, 8), jnp.float32),
            pltpu.VMEM((CHUNK, OUT_D), jnp.float32),
            pltpu.VMEM((16, 16), jnp.float32),
            pltpu.SemaphoreType.DMA,
            pltpu.SemaphoreType.DMA,
        ],
    )
    def body(xt_hbm, ftab_hbm, gs_hbm, out_hbm,
             xyz, ia, ib, ra, rb, ob, gsv, sa, sb):
        wid = lax.axis_index("s") * NC + lax.axis_index("c")
        base_w = wid * PPW
        pltpu.sync_copy(gs_hbm, gsv)
        iota = lax.iota(jnp.int32, 16)
        lane8 = iota * 8
        lane8c = [lane8 + c for c in range(8)]
        iota3 = iota * 3

        def gs_bcast(l):
            return gsv[l]
        idxbufs = [ia, ib]
        rowbufs = [ra, rb]
        sems = [sa, sb]
        zero_f = jnp.float32(0.0)
        one_f = jnp.float32(1.0)

        def load_xyz(g):
            p3 = g * 48 + iota3
            x = plsc.load_gather(xyz, [p3])
            y = plsc.load_gather(xyz, [p3 + 1])
            z = plsc.load_gather(xyz, [p3 + 2])
            return x, y, z

        def corner_hashes(x, y, z, gsl):
            xi, yi, zi = corner_idx(x, y, z, gsl)
            hx = (xi, xi + 1)
            y0 = yi * jnp.int32(PR1)
            hy = (y0, y0 + jnp.int32(PR1))
            z0 = zi * jnp.int32(PR2)
            hz = (z0, z0 + jnp.int32(PR2))
            hs = []
            for c in range(8):
                i, j, k = c >> 2, (c >> 1) & 1, c & 1
                hs.append((hx[i] ^ hy[j] ^ hz[k]) & MASK)
            return xi, yi, zi, hs

        def corner_idx(x, y, z, gsl):
            xc = jnp.minimum(jnp.maximum(x, zero_f), one_f)
            yc = jnp.minimum(jnp.maximum(y, zero_f), one_f)
            zc = jnp.minimum(jnp.maximum(z, zero_f), one_f)
            xi = (xc / gsl).astype(jnp.int32)
            yi = (yc / gsl).astype(jnp.int32)
            zi = (zc / gsl).astype(jnp.int32)
            return xi, yi, zi

        def phase_a(l, g):
            x, y, z = load_xyz(g)
            _, _, _, hs = corner_hashes(x, y, z, gs_bcast(l))
            pos0 = g * 128
            loff = l * TS
            dst = idxbufs[l % 2]
            for c in range(8):
                blk = lax.shift_right_logical(hs[c] + loff, 2)
                plsc.store_scatter(dst, [pos0 + lane8c[c]], blk)

        def phase_c(l, g):
            x, y, z = load_xyz(g)
            gsl = gs_bcast(l)
            xi, yi, zi, hs = corner_hashes(x, y, z, gsl)
            wx = (x - xi.astype(jnp.float32) * gsl) / gsl
            wy = (y - yi.astype(jnp.float32) * gsl) / gsl
            wz = (z - zi.astype(jnp.float32) * gsl) / gsl
            pos0 = g * 128
            rows = rowbufs[l % 2]
            prow = g * 16 + iota
            offs = [(hs[c] & 3) * 2 for c in range(8)]
            for f in range(N_FEATS):
                e = [plsc.load_gather(rows, [pos0 + lane8c[c], offs[c] + f])
                     for c in range(8)]
                c00 = e[0] * (one_f - wx) + e[4] * wx
                c01 = e[1] * (one_f - wx) + e[5] * wx
                c10 = e[2] * (one_f - wx) + e[6] * wx
                c11 = e[3] * (one_f - wx) + e[7] * wx
                c0 = c00 * (one_f - wy) + c10 * wy
                c1 = c01 * (one_f - wy) + c11 * wy
                r = c0 * (one_f - wz) + c1 * wz
                plsc.store_scatter(ob, [prow, jnp.full((16,), 2 * l + f, jnp.int32)], r)

        def chunk_body(ci, carry):
            cb = base_w + ci * CHUNK
            pltpu.sync_copy(xt_hbm.at[pl.ds(cb * 3, CHUNK * 3)], xyz)
            lax.fori_loop(0, G, lambda g, _: phase_a(0, g), None)
            cps = [None] * N_LEVELS
            cps[0] = pltpu.async_copy(ftab_hbm.at[idxbufs[0]], rowbufs[0], sems[0])
            for l in range(N_LEVELS):
                if l + 1 < N_LEVELS:
                    lax.fori_loop(0, G, lambda g, _, l=l: phase_a(l + 1, g), None)
                    cps[l + 1] = pltpu.async_copy(
                        ftab_hbm.at[idxbufs[(l + 1) % 2]],
                        rowbufs[(l + 1) % 2], sems[(l + 1) % 2])
                cps[l].wait()
                lax.fori_loop(0, G, lambda g, _, l=l: phase_c(l, g), None)
            pltpu.sync_copy(ob, out_hbm.at[pl.ds(cb, CHUNK)])
            return carry

        lax.fori_loop(0, NCHUNK, chunk_body, None)

    return body(xt, ftab, gs)


def kernel(x, tables):
    b = jnp.exp((jnp.log(jnp.float32(512.0)) - jnp.log(jnp.float32(16.0)))
                / (N_LEVELS - 1))
    res = jnp.stack([jnp.floor(jnp.float32(16.0) * b ** i)
                     for i in range(N_LEVELS)])
    gs = (jnp.float32(1.0) - jnp.float32(0.0)) / res
    gs_rep = jnp.tile(gs[:, None], (1, 16))
    scale = jnp.float32(1.0) + x[0, 0] * jnp.float32(0.0)
    xt = x.reshape(3 * N_POINTS) * scale
    tview = tables.reshape(N_LEVELS, TS // 128, 128, N_FEATS)
    tview = tview.transpose(0, 1, 3, 2).reshape(TWORDS)
    ftab = _sc_relayout(tview).reshape(TWORDS // 8, 8)
    out = _sc_embed(xt, ftab, gs_rep)
    keep_mask = x == jnp.maximum(jnp.minimum(x, 1.0), 0.0)
    keep_all = jnp.sum(keep_mask, axis=-1) == keep_mask.shape[-1]
    return out, keep_all


# speedup vs baseline: 7.3310x; 1.0005x over previous
import functools

import jax
import jax.numpy as jnp
from jax import lax
from jax.experimental import pallas as pl
from jax.experimental.pallas import tpu as pltpu
from jax.experimental.pallas import tpu_sc as plsc

N_POINTS = 262144
N_LEVELS = 16
N_FEATS = 2
TS = 2 ** 19
MASK = TS - 1
PR1 = -1640531535
PR2 = 805459861
NC, NS = 2, 16
NW = NC * NS
PPW = N_POINTS // NW
CHUNK = 512
NCHUNK = PPW // CHUNK
G = CHUNK // 16
OUT_D = N_LEVELS * N_FEATS


TWORDS = N_LEVELS * TS * N_FEATS
CONV_CHUNK = 16384
CONV_SLAB = TWORDS // NW
CONV_ITERS = CONV_SLAB // CONV_CHUNK


def _sc_relayout(tview):
    mesh = plsc.VectorSubcoreMesh(
        core_axis_name="c", subcore_axis_name="s",
        num_cores=NC, num_subcores=NS)

    @functools.partial(
        pl.kernel,
        out_type=jax.ShapeDtypeStruct((TWORDS,), jnp.float32),
        mesh=mesh,
        compiler_params=pltpu.CompilerParams(
            needs_layout_passes=False, use_tc_tiling_on_sc=False),
        scratch_types=[
            pltpu.VMEM((CONV_CHUNK,), jnp.float32),
            pltpu.VMEM((CONV_CHUNK,), jnp.float32),
            pltpu.VMEM((CONV_CHUNK,), jnp.float32),
            pltpu.VMEM((CONV_CHUNK,), jnp.float32),
            pltpu.SemaphoreType.DMA,
            pltpu.SemaphoreType.DMA,
            pltpu.SemaphoreType.DMA,
            pltpu.SemaphoreType.DMA,
        ],
    )
    def conv(in_hbm, out_hbm, va, vb, oa, ob, sia, sib, soa, sob):
        wid = lax.axis_index("s") * NC + lax.axis_index("c")
        base = wid * CONV_SLAB
        iota = lax.iota(jnp.int32, 16)
        iota2 = iota * 2
        vins = [va, vb]
        vouts = [oa, ob]
        sis = [sia, sib]
        sos = [soa, sob]

        def shuffle(vin, vout):
            def blk(b, carry):
                b256 = b * 256
                for j in range(8):
                    f0 = vin[pl.ds(b256 + j * 16, 16)]
                    f1 = vin[pl.ds(b256 + 128 + j * 16, 16)]
                    dst = b256 + j * 32 + iota2
                    plsc.store_scatter(vout, [dst], f0)
                    plsc.store_scatter(vout, [dst + 1], f1)
                return carry
            lax.fori_loop(0, CONV_CHUNK // 256, blk, None)

        def it(ci, carry):
            off0 = base + 2 * ci * CONV_CHUNK
            cin = [pltpu.async_copy(
                in_hbm.at[pl.ds(off0 + p * CONV_CHUNK, CONV_CHUNK)],
                vins[p], sis[p]) for p in range(2)]
            couts = [None, None]
            for p in range(2):
                cin[p].wait()
                shuffle(vins[p], vouts[p])
                couts[p] = pltpu.async_copy(
                    vouts[p],
                    out_hbm.at[pl.ds(off0 + p * CONV_CHUNK, CONV_CHUNK)],
                    sos[p])
            couts[0].wait()
            couts[1].wait()
            return carry

        lax.fori_loop(0, CONV_ITERS // 2, it, None)

    return conv(tview)


def _sc_embed(xt, ftab, gs):
    mesh = plsc.VectorSubcoreMesh(
        core_axis_name="c", subcore_axis_name="s",
        num_cores=NC, num_subcores=NS)

    @functools.partial(
        pl.kernel,
        out_type=jax.ShapeDtypeStruct((N_POINTS, OUT_D), jnp.float32),
        mesh=mesh,
        compiler_params=pltpu.CompilerParams(
            needs_layout_passes=False, use_tc_tiling_on_sc=False),
        scratch_types=[
            pltpu.VMEM((CHUNK * 3,), jnp.float32),
            pltpu.VMEM((CHUNK * 8,), jnp.int32),
            pltpu.VMEM((CHUNK * 8,), jnp.int32),
            pltpu.VMEM((CHUNK * 8, 8), jnp.float32),
            pltpu.VMEM((CHUNK * 8, 8), jnp.float32),
            pltpu.VMEM((CHUNK, OUT_D), jnp.float32),
            pltpu.VMEM((16, 16), jnp.float32),
            pltpu.SemaphoreType.DMA,
            pltpu.SemaphoreType.DMA,
        ],
    )
    def body(xt_hbm, ftab_hbm, gs_hbm, out_hbm,
             xyz, ia, ib, ra, rb, ob, gsv, sa, sb):
        wid = lax.axis_index("s") * NC + lax.axis_index("c")
        base_w = wid * PPW
        pltpu.sync_copy(gs_hbm, gsv)
        iota = lax.iota(jnp.int32, 16)
        lane8 = iota * 8
        lane8c = [lane8 + c for c in range(8)]
        iota3 = iota * 3

        def gs_bcast(l):
            return gsv[l]
        idxbufs = [ia, ib]
        rowbufs = [ra, rb]
        sems = [sa, sb]
        zero_f = jnp.float32(0.0)
        one_f = jnp.float32(1.0)

        def load_xyz(g):
            p3 = g * 48 + iota3
            x = plsc.load_gather(xyz, [p3])
            y = plsc.load_gather(xyz, [p3 + 1])
            z = plsc.load_gather(xyz, [p3 + 2])
            return x, y, z

        def corner_hashes(x, y, z, gsl):
            xi, yi, zi = corner_idx(x, y, z, gsl)
            hx = (xi, xi + 1)
            y0 = yi * jnp.int32(PR1)
            hy = (y0, y0 + jnp.int32(PR1))
            z0 = zi * jnp.int32(PR2)
            hz = (z0, z0 + jnp.int32(PR2))
            hs = []
            for c in range(8):
                i, j, k = c >> 2, (c >> 1) & 1, c & 1
                hs.append((hx[i] ^ hy[j] ^ hz[k]) & MASK)
            return xi, yi, zi, hs

        def corner_idx(x, y, z, gsl):
            xc = jnp.minimum(jnp.maximum(x, zero_f), one_f)
            yc = jnp.minimum(jnp.maximum(y, zero_f), one_f)
            zc = jnp.minimum(jnp.maximum(z, zero_f), one_f)
            xi = (xc / gsl).astype(jnp.int32)
            yi = (yc / gsl).astype(jnp.int32)
            zi = (zc / gsl).astype(jnp.int32)
            return xi, yi, zi

        def phase_a(l, g):
            x, y, z = load_xyz(g)
            _, _, _, hs = corner_hashes(x, y, z, gs_bcast(l))
            pos0 = g * 128
            loff = l * TS
            dst = idxbufs[l % 2]
            for c in range(8):
                blk = lax.shift_right_logical(hs[c] + loff, 2)
                plsc.store_scatter(dst, [pos0 + lane8c[c]], blk)

        def phase_c(l, g):
            x, y, z = load_xyz(g)
            gsl = gs_bcast(l)
            xi, yi, zi, hs = corner_hashes(x, y, z, gsl)
            wx = (x - xi.astype(jnp.float32) * gsl) / gsl
            wy = (y - yi.astype(jnp.float32) * gsl) / gsl
            wz = (z - zi.astype(jnp.float32) * gsl) / gsl
            pos0 = g * 128
            rows = rowbufs[l % 2]
            prow = g * 16 + iota
            offs = [(hs[c] & 3) * 2 for c in range(8)]
            for f in range(N_FEATS):
                e = [plsc.load_gather(rows, [pos0 + lane8c[c], offs[c] + f])
                     for c in range(8)]
                c00 = e[0] * (one_f - wx) + e[4] * wx
                c01 = e[1] * (one_f - wx) + e[5] * wx
                c10 = e[2] * (one_f - wx) + e[6] * wx
                c11 = e[3] * (one_f - wx) + e[7] * wx
                c0 = c00 * (one_f - wy) + c10 * wy
                c1 = c01 * (one_f - wy) + c11 * wy
                r = c0 * (one_f - wz) + c1 * wz
                plsc.store_scatter(ob, [prow, jnp.full((16,), 2 * l + f, jnp.int32)], r)

        def chunk_body(ci, carry):
            cb = base_w + ci * CHUNK
            pltpu.sync_copy(xt_hbm.at[pl.ds(cb * 3, CHUNK * 3)], xyz)
            lax.fori_loop(0, G, lambda g, _: phase_a(0, g), None)
            cps = [None] * N_LEVELS
            cps[0] = pltpu.async_copy(ftab_hbm.at[idxbufs[0]], rowbufs[0], sems[0])
            for l in range(N_LEVELS):
                if l + 1 < N_LEVELS:
                    lax.fori_loop(0, G, lambda g, _, l=l: phase_a(l + 1, g), None)
                    cps[l + 1] = pltpu.async_copy(
                        ftab_hbm.at[idxbufs[(l + 1) % 2]],
                        rowbufs[(l + 1) % 2], sems[(l + 1) % 2])
                cps[l].wait()
                lax.fori_loop(0, G, lambda g, _, l=l: phase_c(l, g), None)
            pltpu.sync_copy(ob, out_hbm.at[pl.ds(cb, CHUNK)])
            return carry

        lax.fori_loop(0, NCHUNK, chunk_body, None)

    return body(xt, ftab, gs)


def kernel(x, tables):
    b = jnp.exp((jnp.log(jnp.float32(512.0)) - jnp.log(jnp.float32(16.0)))
                / (N_LEVELS - 1))
    res = jnp.stack([jnp.floor(jnp.float32(16.0) * b ** i)
                     for i in range(N_LEVELS)])
    gs = (jnp.float32(1.0) - jnp.float32(0.0)) / res
    gs_rep = jnp.tile(gs[:, None], (1, 16))
    scale = jnp.float32(1.0) + x[0, 0] * jnp.float32(0.0)
    xt = x.reshape(3 * N_POINTS) * scale
    tview = tables.reshape(N_LEVELS, TS // 128, 128, N_FEATS)
    tview = tview.transpose(0, 1, 3, 2).reshape(TWORDS)
    ftab = _sc_relayout(tview).reshape(TWORDS // 8, 8)
    out = _sc_embed(xt, ftab, gs_rep)
    keep_mask = x == jnp.maximum(jnp.minimum(x, 1.0), 0.0)
    keep_all = jnp.sum(keep_mask, axis=-1) == keep_mask.shape[-1]
    return out, keep_all
